# baseline (speedup 1.0000x reference)
import functools

import jax
import jax.numpy as jnp
from jax import lax
from jax.experimental import pallas as pl
from jax.experimental.pallas import tpu as pltpu
from jax.experimental.pallas import tpu_sc as plsc

N_NODES = 10000
N_EDGES = 320000
D_FEAT = 128

_NC = 2
_NS = 16
_NW = _NC * _NS
_EPW = N_EDGES // _NW
_L = 16

_CW = 10240
_GRP = _CW // _L
_TAIL_C0 = N_EDGES - _CW
_TAIL_LO = (_NW - 1) * _CW - _TAIL_C0

def _sc_mesh():
    return plsc.VectorSubcoreMesh(
        core_axis_name="c", subcore_axis_name="s", num_cores=_NC, num_subcores=_NS
    )


def _chunk_start_and_lo(wid):
    c0 = jnp.where(wid == _NW - 1, _TAIL_C0, wid * _CW)
    c0 = pl.multiple_of(c0, 128)
    lo = jnp.where(wid == _NW - 1, _TAIL_LO, 0)
    return c0, lo


def _deg_body(ei_hbm, z_hbm, out_hbm, ei_v, acc_v, sem):
    wid = lax.axis_index("s") * _NC + lax.axis_index("c")
    c0, lo = _chunk_start_and_lo(wid)
    cpz = pltpu.async_copy(z_hbm.at[0], acc_v, sem)
    cp = pltpu.async_copy(ei_hbm.at[:, pl.ds(c0, _CW)], ei_v, sem)
    cpz.wait()
    cp.wait()

    ones = jnp.ones((_L,), jnp.float32)
    lane = lax.iota(jnp.int32, _L)

    @plsc.parallel_loop(0, _GRP, unroll=4)
    def _count(i):
        m = (i * _L + lane) >= lo
        d = ei_v[1, pl.ds(i * _L, _L)]
        plsc.addupdate_scatter(acc_v, [d], ones, mask=m)

    pltpu.sync_copy(acc_v, out_hbm.at[wid])


@functools.cache
def _deg_kernel():
    return pl.kernel(
        _deg_body,
        out_type=jax.ShapeDtypeStruct((_NW, N_NODES), jnp.float32),
        mesh=_sc_mesh(),
        scratch_types=[
            pltpu.VMEM((2, _CW), jnp.int32),
            pltpu.VMEM((N_NODES,), jnp.float32),
            pltpu.SemaphoreType.DMA,
        ],
        name="sc_deg",
        compiler_params=pltpu.CompilerParams(needs_layout_passes=False),
    )


def _agg_body(ei_hbm, w_hbm, z_hbm, out_hbm, ei_v, w_v, acc_v, sem):
    wid = lax.axis_index("s") * _NC + lax.axis_index("c")
    c0, lo = _chunk_start_and_lo(wid)
    cpz = pltpu.async_copy(z_hbm.at[0], acc_v, sem)
    cp1 = pltpu.async_copy(ei_hbm.at[:, pl.ds(c0, _CW)], ei_v, sem)
    cp2 = pltpu.async_copy(w_hbm.at[0], w_v, sem)
    cpz.wait()
    cp1.wait()
    cp2.wait()

    lane = lax.iota(jnp.int32, _L)

    @plsc.parallel_loop(0, _GRP, unroll=4)
    def _edge(i):
        m = (i * _L + lane) >= lo
        s = ei_v[0, pl.ds(i * _L, _L)]
        d = ei_v[1, pl.ds(i * _L, _L)]
        vals = plsc.load_gather(w_v, [s], mask=m)
        plsc.addupdate_scatter(acc_v, [d], vals, mask=m)

    pltpu.sync_copy(acc_v, out_hbm.at[wid])


@functools.cache
def _agg_kernel():
    return pl.kernel(
        _agg_body,
        out_type=jax.ShapeDtypeStruct((_NW, N_NODES), jnp.float32),
        mesh=_sc_mesh(),
        scratch_types=[
            pltpu.VMEM((2, _CW), jnp.int32),
            pltpu.VMEM((N_NODES,), jnp.float32),
            pltpu.VMEM((N_NODES,), jnp.float32),
            pltpu.SemaphoreType.DMA,
        ],
        compiler_params=pltpu.CompilerParams(needs_layout_passes=False),
    )


def _mm_body(x_ref, w_ref_in, ws_ref_in, y_ref, z_ref):
    wc = jnp.concatenate([w_ref_in[...], ws_ref_in[...]], axis=1)
    y2t = lax.dot_general(
        wc, x_ref[...], (((0,), (1,)), ((), ())),
        preferred_element_type=jnp.float32,
    )
    y_ref[...] = y2t[0:1, :]
    z_ref[...] = y2t[1:2, :]


_mm_call = pl.pallas_call(
    _mm_body,
    out_shape=(
        jax.ShapeDtypeStruct((1, N_NODES), jnp.float32),
        jax.ShapeDtypeStruct((1, N_NODES), jnp.float32),
    ),
)


def _norm_body(degp_ref, y_ref, w_ref, norm_ref):
    deg = jnp.sum(degp_ref[...], axis=0, keepdims=True)
    norm = 1.0 / jnp.sqrt(jnp.clip(deg, 1.0, None))
    norm_ref[...] = norm
    w_ref[...] = y_ref[...] * norm


_norm_call = pl.pallas_call(
    _norm_body,
    out_shape=(
        jax.ShapeDtypeStruct((1, N_NODES), jnp.float32),
        jax.ShapeDtypeStruct((1, N_NODES), jnp.float32),
    ),
)


def _fin_body(aggp_ref, norm_ref, z_ref, b_ref, g_ref, act_ref, lp_ref):
    agg = jnp.sum(aggp_ref[...], axis=0, keepdims=True)
    pred = norm_ref[...] * agg + z_ref[...] + b_ref[0, 0]
    m = jnp.max(pred)
    s = jnp.sum(jnp.exp(pred - m))
    t = pred + g_ref[...]
    tm = jnp.max(t)
    idx = lax.broadcasted_iota(jnp.int32, t.shape, 1)
    act = jnp.min(jnp.where(t == tm, idx, jnp.int32(N_NODES)))
    act_ref[...] = jnp.reshape(act, (1, 1))
    pred_at = jnp.sum(jnp.where(idx == act, pred, 0.0))
    lp_ref[...] = jnp.reshape(pred_at - m - jnp.log(s), (1, 1))


_fin_call = pl.pallas_call(
    _fin_body,
    out_shape=(
        jax.ShapeDtypeStruct((1, 1), jnp.int32),
        jax.ShapeDtypeStruct((1, 1), jnp.float32),
    ),
)


def _np_threefry2x32(k1, k2, x0, x1):
    import numpy as np

    def rotl(v, d):
        return (v << np.uint32(d)) | (v >> np.uint32(32 - d))

    rot_a = (13, 15, 26, 6)
    rot_b = (17, 29, 16, 24)
    ks = [k1, k2, np.uint32(k1 ^ k2 ^ np.uint32(0x1BD11BDA))]
    x = [x0 + ks[0], x1 + ks[1]]

    def rounds(x, rots):
        for r in rots:
            x[0] = x[0] + x[1]
            x[1] = x[0] ^ rotl(x[1], r)
        return x

    x = rounds(x, rot_a)
    x[0] += ks[1]
    x[1] += ks[2] + np.uint32(1)
    x = rounds(x, rot_b)
    x[0] += ks[2]
    x[1] += ks[0] + np.uint32(2)
    x = rounds(x, rot_a)
    x[0] += ks[0]
    x[1] += ks[1] + np.uint32(3)
    x = rounds(x, rot_b)
    x[0] += ks[1]
    x[1] += ks[2] + np.uint32(4)
    x = rounds(x, rot_a)
    x[0] += ks[2]
    x[1] += ks[0] + np.uint32(5)
    return x


@functools.cache
def _gumbel_const():
    import numpy as np

    old = np.seterr(over="ignore")
    try:
        k1, k2 = np.uint32(0), np.uint32(42)
        idx = np.arange(N_NODES, dtype=np.uint64)
        c1 = (idx >> np.uint64(32)).astype(np.uint32)
        c2 = (idx & np.uint64(0xFFFFFFFF)).astype(np.uint32)
        b1, b2 = _np_threefry2x32(k1, k2, c1, c2)
        bits = b1 ^ b2
        float_bits = (bits >> np.uint32(9)) | np.uint32(0x3F800000)
        floats = float_bits.view(np.float32) - np.float32(1.0)
        tiny = np.float32(np.finfo(np.float32).tiny)
        span = np.float32(np.float32(1.0) - tiny)
        u = np.maximum(tiny, floats * span + tiny)
        g = -np.log(-np.log(u))
    finally:
        np.seterr(**old)
    return g.astype(np.float32).reshape(1, N_NODES)


def kernel(x, edge_index, batch, W, W_self, b):
    gumbel = jnp.asarray(_gumbel_const())

    zeros2d = jnp.zeros((1, N_NODES), jnp.float32)
    degp = _deg_kernel()(edge_index, zeros2d)
    y, z = _mm_call(x, W, W_self)
    wvec, norm = _norm_call(degp, y)
    aggp = _agg_kernel()(edge_index, wvec, zeros2d)
    act, lp = _fin_call(aggp, norm, z, jnp.reshape(b, (1, 1)), gumbel)

    action_index = jnp.reshape(act, (1,))
    log_prob = lp
    return action_index, log_prob


# speedup vs baseline: 1.1016x; 1.1016x over previous
import functools

import jax
import jax.numpy as jnp
from jax import lax
from jax.experimental import pallas as pl
from jax.experimental.pallas import tpu as pltpu
from jax.experimental.pallas import tpu_sc as plsc

N_NODES = 10000
N_EDGES = 320000
D_FEAT = 128

_NC = 2
_NS = 16
_NW = _NC * _NS
_EPW = N_EDGES // _NW
_L = 16

_CW = 10240
_GRP = _CW // _L
_TAIL_C0 = N_EDGES - _CW
_TAIL_LO = (_NW - 1) * _CW - _TAIL_C0

def _sc_mesh():
    return plsc.VectorSubcoreMesh(
        core_axis_name="c", subcore_axis_name="s", num_cores=_NC, num_subcores=_NS
    )


def _chunk_start_and_lo(wid):
    c0 = jnp.where(wid == _NW - 1, _TAIL_C0, wid * _CW)
    c0 = pl.multiple_of(c0, 128)
    lo = jnp.where(wid == _NW - 1, _TAIL_LO, 0)
    return c0, lo


def _deg_body(ei_hbm, out_hbm, ei_v, acc_v, sem):
    wid = lax.axis_index("s") * _NC + lax.axis_index("c")
    c0, lo = _chunk_start_and_lo(wid)
    cp = pltpu.async_copy(ei_hbm.at[:, pl.ds(c0, _CW)], ei_v, sem)

    zeros = jnp.zeros((_L,), jnp.float32)

    @plsc.parallel_loop(0, N_NODES // _L, unroll=4)
    def _zero(i):
        acc_v[pl.ds(i * _L, _L)] = zeros

    cp.wait()

    ones = jnp.ones((_L,), jnp.float32)
    lane = lax.iota(jnp.int32, _L)

    @plsc.parallel_loop(0, _GRP, unroll=4)
    def _count(i):
        m = (i * _L + lane) >= lo
        d = ei_v[1, pl.ds(i * _L, _L)]
        plsc.addupdate_scatter(acc_v, [d], ones, mask=m)

    pltpu.sync_copy(acc_v, out_hbm.at[wid])


@functools.cache
def _deg_kernel():
    return pl.kernel(
        _deg_body,
        out_type=jax.ShapeDtypeStruct((_NW, N_NODES), jnp.float32),
        mesh=_sc_mesh(),
        scratch_types=[
            pltpu.VMEM((2, _CW), jnp.int32),
            pltpu.VMEM((N_NODES,), jnp.float32),
            pltpu.SemaphoreType.DMA,
        ],
        name="sc_deg",
        compiler_params=pltpu.CompilerParams(needs_layout_passes=False),
    )


def _agg_body(ei_hbm, w_hbm, out_hbm, ei_v, w_v, acc_v, sem):
    wid = lax.axis_index("s") * _NC + lax.axis_index("c")
    c0, lo = _chunk_start_and_lo(wid)
    cp1 = pltpu.async_copy(ei_hbm.at[:, pl.ds(c0, _CW)], ei_v, sem)
    cp2 = pltpu.async_copy(w_hbm.at[0], w_v, sem)

    zeros = jnp.zeros((_L,), jnp.float32)

    @plsc.parallel_loop(0, N_NODES // _L, unroll=4)
    def _zero(i):
        acc_v[pl.ds(i * _L, _L)] = zeros

    cp1.wait()
    cp2.wait()

    lane = lax.iota(jnp.int32, _L)

    @plsc.parallel_loop(0, _GRP, unroll=4)
    def _edge(i):
        m = (i * _L + lane) >= lo
        s = ei_v[0, pl.ds(i * _L, _L)]
        d = ei_v[1, pl.ds(i * _L, _L)]
        vals = plsc.load_gather(w_v, [s], mask=m)
        plsc.addupdate_scatter(acc_v, [d], vals, mask=m)

    pltpu.sync_copy(acc_v, out_hbm.at[wid])


@functools.cache
def _agg_kernel():
    return pl.kernel(
        _agg_body,
        out_type=jax.ShapeDtypeStruct((_NW, N_NODES), jnp.float32),
        mesh=_sc_mesh(),
        scratch_types=[
            pltpu.VMEM((2, _CW), jnp.int32),
            pltpu.VMEM((N_NODES,), jnp.float32),
            pltpu.VMEM((N_NODES,), jnp.float32),
            pltpu.SemaphoreType.DMA,
        ],
        compiler_params=pltpu.CompilerParams(needs_layout_passes=False),
    )


def _mm_body(x_ref, w_ref_in, ws_ref_in, y_ref, z_ref):
    wc = jnp.concatenate([w_ref_in[...], ws_ref_in[...]], axis=1)
    y2t = lax.dot_general(
        wc, x_ref[...], (((0,), (1,)), ((), ())),
        preferred_element_type=jnp.float32,
    )
    y_ref[...] = y2t[0:1, :]
    z_ref[...] = y2t[1:2, :]


_mm_call = pl.pallas_call(
    _mm_body,
    out_shape=(
        jax.ShapeDtypeStruct((1, N_NODES), jnp.float32),
        jax.ShapeDtypeStruct((1, N_NODES), jnp.float32),
    ),
)


def _norm_body(degp_ref, y_ref, w_ref, norm_ref):
    deg = jnp.sum(degp_ref[...], axis=0, keepdims=True)
    norm = 1.0 / jnp.sqrt(jnp.clip(deg, 1.0, None))
    norm_ref[...] = norm
    w_ref[...] = y_ref[...] * norm


_norm_call = pl.pallas_call(
    _norm_body,
    out_shape=(
        jax.ShapeDtypeStruct((1, N_NODES), jnp.float32),
        jax.ShapeDtypeStruct((1, N_NODES), jnp.float32),
    ),
)


def _fin_body(aggp_ref, norm_ref, z_ref, b_ref, g_ref, act_ref, lp_ref):
    agg = jnp.sum(aggp_ref[...], axis=0, keepdims=True)
    pred = norm_ref[...] * agg + z_ref[...] + b_ref[0, 0]
    m = jnp.max(pred)
    s = jnp.sum(jnp.exp(pred - m))
    t = pred + g_ref[...]
    tm = jnp.max(t)
    idx = lax.broadcasted_iota(jnp.int32, t.shape, 1)
    act = jnp.min(jnp.where(t == tm, idx, jnp.int32(N_NODES)))
    act_ref[...] = jnp.reshape(act, (1, 1))
    pred_at = jnp.sum(jnp.where(idx == act, pred, 0.0))
    lp_ref[...] = jnp.reshape(pred_at - m - jnp.log(s), (1, 1))


_fin_call = pl.pallas_call(
    _fin_body,
    out_shape=(
        jax.ShapeDtypeStruct((1, 1), jnp.int32),
        jax.ShapeDtypeStruct((1, 1), jnp.float32),
    ),
)


def _np_threefry2x32(k1, k2, x0, x1):
    import numpy as np

    def rotl(v, d):
        return (v << np.uint32(d)) | (v >> np.uint32(32 - d))

    rot_a = (13, 15, 26, 6)
    rot_b = (17, 29, 16, 24)
    ks = [k1, k2, np.uint32(k1 ^ k2 ^ np.uint32(0x1BD11BDA))]
    x = [x0 + ks[0], x1 + ks[1]]

    def rounds(x, rots):
        for r in rots:
            x[0] = x[0] + x[1]
            x[1] = x[0] ^ rotl(x[1], r)
        return x

    x = rounds(x, rot_a)
    x[0] += ks[1]
    x[1] += ks[2] + np.uint32(1)
    x = rounds(x, rot_b)
    x[0] += ks[2]
    x[1] += ks[0] + np.uint32(2)
    x = rounds(x, rot_a)
    x[0] += ks[0]
    x[1] += ks[1] + np.uint32(3)
    x = rounds(x, rot_b)
    x[0] += ks[1]
    x[1] += ks[2] + np.uint32(4)
    x = rounds(x, rot_a)
    x[0] += ks[2]
    x[1] += ks[0] + np.uint32(5)
    return x


@functools.cache
def _gumbel_const():
    import numpy as np

    old = np.seterr(over="ignore")
    try:
        k1, k2 = np.uint32(0), np.uint32(42)
        idx = np.arange(N_NODES, dtype=np.uint64)
        c1 = (idx >> np.uint64(32)).astype(np.uint32)
        c2 = (idx & np.uint64(0xFFFFFFFF)).astype(np.uint32)
        b1, b2 = _np_threefry2x32(k1, k2, c1, c2)
        bits = b1 ^ b2
        float_bits = (bits >> np.uint32(9)) | np.uint32(0x3F800000)
        floats = float_bits.view(np.float32) - np.float32(1.0)
        tiny = np.float32(np.finfo(np.float32).tiny)
        span = np.float32(np.float32(1.0) - tiny)
        u = np.maximum(tiny, floats * span + tiny)
        g = -np.log(-np.log(u))
    finally:
        np.seterr(**old)
    return g.astype(np.float32).reshape(1, N_NODES)


def kernel(x, edge_index, batch, W, W_self, b):
    gumbel = jnp.asarray(_gumbel_const())

    degp = _deg_kernel()(edge_index)
    y, z = _mm_call(x, W, W_self)
    wvec, norm = _norm_call(degp, y)
    aggp = _agg_kernel()(edge_index, wvec)
    act, lp = _fin_call(aggp, norm, z, jnp.reshape(b, (1, 1)), gumbel)

    action_index = jnp.reshape(act, (1,))
    log_prob = lp
    return action_index, log_prob


# speedup vs baseline: 1.1026x; 1.0009x over previous
import functools

import jax
import jax.numpy as jnp
from jax import lax
from jax.experimental import pallas as pl
from jax.experimental.pallas import tpu as pltpu
from jax.experimental.pallas import tpu_sc as plsc

N_NODES = 10000
N_EDGES = 320000
D_FEAT = 128

_NC = 2
_NS = 16
_NW = _NC * _NS
_EPW = N_EDGES // _NW
_L = 16

_CW = 10240
_GRP = _CW // _L
_TAIL_C0 = N_EDGES - _CW
_TAIL_LO = (_NW - 1) * _CW - _TAIL_C0

def _sc_mesh():
    return plsc.VectorSubcoreMesh(
        core_axis_name="c", subcore_axis_name="s", num_cores=_NC, num_subcores=_NS
    )


def _chunk_start_and_lo(wid):
    c0 = jnp.where(wid == _NW - 1, _TAIL_C0, wid * _CW)
    c0 = pl.multiple_of(c0, 128)
    lo = jnp.where(wid == _NW - 1, _TAIL_LO, 0)
    return c0, lo


def _deg_body(ei_hbm, out_hbm, ei_v, acc_v, sem):
    wid = lax.axis_index("s") * _NC + lax.axis_index("c")
    c0, lo = _chunk_start_and_lo(wid)
    cp = pltpu.async_copy(ei_hbm.at[:, pl.ds(c0, _CW)], ei_v, sem)

    zeros = jnp.zeros((_L,), jnp.float32)

    @plsc.parallel_loop(0, N_NODES // _L, unroll=4)
    def _zero(i):
        acc_v[pl.ds(i * _L, _L)] = zeros

    cp.wait()

    ones = jnp.ones((_L,), jnp.float32)
    lane = lax.iota(jnp.int32, _L)

    @plsc.parallel_loop(0, _GRP, unroll=4)
    def _count(i):
        m = (i * _L + lane) >= lo
        d = ei_v[1, pl.ds(i * _L, _L)]
        plsc.addupdate_scatter(acc_v, [d], ones, mask=m)

    pltpu.sync_copy(acc_v, out_hbm.at[wid])


@functools.cache
def _deg_kernel():
    return pl.kernel(
        _deg_body,
        out_type=jax.ShapeDtypeStruct((_NW, N_NODES), jnp.float32),
        mesh=_sc_mesh(),
        scratch_types=[
            pltpu.VMEM((2, _CW), jnp.int32),
            pltpu.VMEM((N_NODES,), jnp.float32),
            pltpu.SemaphoreType.DMA,
        ],
        name="sc_deg",
        compiler_params=pltpu.CompilerParams(needs_layout_passes=False),
    )


def _agg_body(ei_hbm, w_hbm, out_hbm, ei_v, w_v, acc_v, sem):
    wid = lax.axis_index("s") * _NC + lax.axis_index("c")
    c0, lo = _chunk_start_and_lo(wid)
    cp1 = pltpu.async_copy(ei_hbm.at[:, pl.ds(c0, _CW)], ei_v, sem)
    cp2 = pltpu.async_copy(w_hbm.at[0], w_v, sem)

    zeros = jnp.zeros((_L,), jnp.float32)

    @plsc.parallel_loop(0, N_NODES // _L, unroll=4)
    def _zero(i):
        acc_v[pl.ds(i * _L, _L)] = zeros

    cp1.wait()
    cp2.wait()

    lane = lax.iota(jnp.int32, _L)

    @plsc.parallel_loop(0, _GRP, unroll=4)
    def _edge(i):
        m = (i * _L + lane) >= lo
        s = ei_v[0, pl.ds(i * _L, _L)]
        d = ei_v[1, pl.ds(i * _L, _L)]
        vals = plsc.load_gather(w_v, [s], mask=m)
        plsc.addupdate_scatter(acc_v, [d], vals, mask=m)

    pltpu.sync_copy(acc_v, out_hbm.at[wid])


@functools.cache
def _agg_kernel():
    return pl.kernel(
        _agg_body,
        out_type=jax.ShapeDtypeStruct((_NW, N_NODES), jnp.float32),
        mesh=_sc_mesh(),
        scratch_types=[
            pltpu.VMEM((2, _CW), jnp.int32),
            pltpu.VMEM((N_NODES,), jnp.float32),
            pltpu.VMEM((N_NODES,), jnp.float32),
            pltpu.SemaphoreType.DMA,
        ],
        compiler_params=pltpu.CompilerParams(needs_layout_passes=False),
    )


def _mm_body(x_ref, w_ref_in, ws_ref_in, y_ref, z_ref):
    wc = jnp.concatenate([w_ref_in[...], ws_ref_in[...]], axis=1)
    y2t = lax.dot_general(
        wc, x_ref[...], (((0,), (1,)), ((), ())),
        preferred_element_type=jnp.float32,
    )
    y_ref[...] = y2t[0:1, :]
    z_ref[...] = y2t[1:2, :]


_mm_call = pl.pallas_call(
    _mm_body,
    out_shape=(
        jax.ShapeDtypeStruct((1, N_NODES), jnp.float32),
        jax.ShapeDtypeStruct((1, N_NODES), jnp.float32),
    ),
)


def _norm_body(degp_ref, y_ref, w_ref, norm_ref):
    deg = jnp.sum(degp_ref[...], axis=0, keepdims=True)
    norm = 1.0 / jnp.sqrt(jnp.clip(deg, 1.0, None))
    norm_ref[...] = norm
    w_ref[...] = y_ref[...] * norm


_norm_call = pl.pallas_call(
    _norm_body,
    out_shape=(
        jax.ShapeDtypeStruct((1, N_NODES), jnp.float32),
        jax.ShapeDtypeStruct((1, N_NODES), jnp.float32),
    ),
)


def _fin_body(aggp_ref, norm_ref, z_ref, b_ref, g_ref, act_ref, lp_ref):
    agg = jnp.sum(aggp_ref[...], axis=0, keepdims=True)
    pred = norm_ref[...] * agg + z_ref[...] + b_ref[0, 0]
    m = jnp.max(pred)
    s = jnp.sum(jnp.exp(pred - m))
    t = pred + g_ref[...]
    tm = jnp.max(t)
    idx = lax.broadcasted_iota(jnp.int32, t.shape, 1)
    act = jnp.min(jnp.where(t == tm, idx, jnp.int32(N_NODES)))
    act_ref[...] = jnp.reshape(act, (1, 1))
    pred_at = jnp.sum(jnp.where(idx == act, pred, 0.0))
    lp_ref[...] = jnp.reshape(pred_at - m - jnp.log(s), (1, 1))


_fin_call = pl.pallas_call(
    _fin_body,
    out_shape=(
        jax.ShapeDtypeStruct((1, 1), jnp.int32),
        jax.ShapeDtypeStruct((1, 1), jnp.float32),
    ),
)


def _np_threefry2x32(k1, k2, x0, x1):
    import numpy as np

    def rotl(v, d):
        return (v << np.uint32(d)) | (v >> np.uint32(32 - d))

    rot_a = (13, 15, 26, 6)
    rot_b = (17, 29, 16, 24)
    ks = [k1, k2, np.uint32(k1 ^ k2 ^ np.uint32(0x1BD11BDA))]
    x = [x0 + ks[0], x1 + ks[1]]

    def rounds(x, rots):
        for r in rots:
            x[0] = x[0] + x[1]
            x[1] = x[0] ^ rotl(x[1], r)
        return x

    x = rounds(x, rot_a)
    x[0] += ks[1]
    x[1] += ks[2] + np.uint32(1)
    x = rounds(x, rot_b)
    x[0] += ks[2]
    x[1] += ks[0] + np.uint32(2)
    x = rounds(x, rot_a)
    x[0] += ks[0]
    x[1] += ks[1] + np.uint32(3)
    x = rounds(x, rot_b)
    x[0] += ks[1]
    x[1] += ks[2] + np.uint32(4)
    x = rounds(x, rot_a)
    x[0] += ks[2]
    x[1] += ks[0] + np.uint32(5)
    return x


@functools.cache
def _gumbel_const():
    import numpy as np

    old = np.seterr(over="ignore")
    try:
        k1, k2 = np.uint32(0), np.uint32(42)
        idx = np.arange(N_NODES, dtype=np.uint64)
        c1 = (idx >> np.uint64(32)).astype(np.uint32)
        c2 = (idx & np.uint64(0xFFFFFFFF)).astype(np.uint32)
        b1, b2 = _np_threefry2x32(k1, k2, c1, c2)
        bits = b1 ^ b2
        float_bits = (bits >> np.uint32(9)) | np.uint32(0x3F800000)
        floats = float_bits.view(np.float32) - np.float32(1.0)
        tiny = np.float32(np.finfo(np.float32).tiny)
        span = np.float32(np.float32(1.0) - tiny)
        u = np.maximum(tiny, floats * span + tiny)
        g = -np.log(-np.log(u))
    finally:
        np.seterr(**old)
    return g.astype(np.float32).reshape(1, N_NODES)


def kernel(x, edge_index, batch, W, W_self, b):
    gumbel = jnp.asarray(_gumbel_const())

    y, z = _mm_call(x, W, W_self)
    degp = _deg_kernel()(edge_index)
    wvec, norm = _norm_call(degp, y)
    aggp = _agg_kernel()(edge_index, wvec)
    act, lp = _fin_call(aggp, norm, z, jnp.reshape(b, (1, 1)), gumbel)

    action_index = jnp.reshape(act, (1,))
    log_prob = lp
    return action_index, log_prob
